# Initial kernel scaffold; baseline (speedup 1.0000x reference)
import functools

import jax
import jax.numpy as jnp
from jax import lax
from jax.experimental import pallas as pl
from jax.experimental.pallas import tpu as pltpu
from jax.experimental.pallas import tpu_sc as plsc

F32 = jnp.float32
NC, NS = 2, 16
NW = NC * NS
BLK = 128
DD = 16

N_S, E_S = 2000, 32000
N_L, E_L = 10000, 320000
NP_S, NP_L = 2048, 10048
KS = 8
KL = 79

_MESH = plsc.VectorSubcoreMesh(core_axis_name="c", subcore_axis_name="s")


def _pad_idx(idx, fill, K, B):
    n = NW * K * B
    idx = idx.astype(jnp.int32)
    pad = jnp.full((n - idx.shape[0],), fill, jnp.int32)
    return jnp.concatenate([idx, pad]).reshape(NW, K, B)


@functools.lru_cache(maxsize=None)
def _sc_gather_multi(specs):
    n_ops = len(specs)
    out_type = [jax.ShapeDtypeStruct((NW * K * B, D), F32) for (_, D, K, B) in specs]
    scratch = []
    for (_, D, K, B) in specs:
        scratch += [pltpu.VMEM((K, B), jnp.int32), pltpu.VMEM((B, D), F32)]
    scratch.append(pltpu.SemaphoreType.DMA)

    def body(*refs):
        ins = refs[:2 * n_ops]
        outs = refs[2 * n_ops:3 * n_ops]
        scr = refs[3 * n_ops:]
        sem = scr[-1]
        cid = lax.axis_index("c")
        sid = lax.axis_index("s")
        wid = cid * NS + sid
        for i, (_, D, K, B) in enumerate(specs):
            table_h, idx_h = ins[2 * i], ins[2 * i + 1]
            out_h = outs[i]
            idx_v, rows_v = scr[2 * i], scr[2 * i + 1]
            pltpu.sync_copy(idx_h.at[wid], idx_v)

            def step(j, _, table_h=table_h, out_h=out_h, idx_v=idx_v,
                     rows_v=rows_v, K=K, B=B):
                pltpu.async_copy(table_h.at[idx_v.at[j]], rows_v, sem).wait()
                pltpu.sync_copy(rows_v, out_h.at[pl.ds((wid * K + j) * B, B)])
                return 0

            lax.fori_loop(0, K, step, 0)

    return pl.kernel(body, out_type=out_type, mesh=_MESH, scratch_types=scratch)


@functools.lru_cache(maxsize=None)
def _sc_segsum(D, NP, K):
    out_type = jax.ShapeDtypeStruct((NC, NP, D), F32)
    stripe = NP // NS
    scratch = [pltpu.VMEM((K, BLK), jnp.int32), pltpu.VMEM((K, BLK), jnp.int32),
               pltpu.VMEM((BLK, D), F32),
               pltpu.VMEM_SHARED((NP, D), F32),
               pltpu.SemaphoreType.DMA]

    def body(table_h, src_h, dst_h, zero_h, out_h, sidx, didx, rows, acc, sem):
        cid = lax.axis_index("c")
        sid = lax.axis_index("s")
        wid = cid * NS + sid
        sl = pl.ds(sid * stripe, stripe)
        pltpu.sync_copy(zero_h.at[sl], acc.at[sl])
        pltpu.sync_copy(src_h.at[wid], sidx)
        pltpu.sync_copy(dst_h.at[wid], didx)
        plsc.subcore_barrier()

        def step(j, _):
            pltpu.async_copy(table_h.at[sidx.at[j]], rows, sem).wait()
            pltpu.sync_copy(rows, acc.at[didx.at[j]], add=True)
            return 0

        lax.fori_loop(0, K, step, 0)
        plsc.subcore_barrier()
        pltpu.sync_copy(acc.at[sl], out_h.at[cid, sl])

    return pl.kernel(body, out_type=out_type, mesh=_MESH, scratch_types=scratch)


@functools.lru_cache(maxsize=None)
def _sc_scatter_add(D, NP, K):
    out_type = jax.ShapeDtypeStruct((NC, NP, D), F32)
    stripe = NP // NS
    scratch = [pltpu.VMEM((K, BLK), jnp.int32),
               pltpu.VMEM((BLK, D), F32),
               pltpu.VMEM_SHARED((NP, D), F32)]

    def body(vals_h, dst_h, zero_h, out_h, didx, rows, acc):
        cid = lax.axis_index("c")
        sid = lax.axis_index("s")
        wid = cid * NS + sid
        sl = pl.ds(sid * stripe, stripe)
        pltpu.sync_copy(zero_h.at[sl], acc.at[sl])
        pltpu.sync_copy(dst_h.at[wid], didx)
        plsc.subcore_barrier()

        def step(j, _):
            pltpu.sync_copy(vals_h.at[pl.ds((wid * K + j) * BLK, BLK)], rows)
            pltpu.sync_copy(rows, acc.at[didx.at[j]], add=True)
            return 0

        lax.fori_loop(0, K, step, 0)
        plsc.subcore_barrier()
        pltpu.sync_copy(acc.at[sl], out_h.at[cid, sl])

    return pl.kernel(body, out_type=out_type, mesh=_MESH, scratch_types=scratch)


@functools.lru_cache(maxsize=None)
def _sc_degree(NP, K):
    out_type = jax.ShapeDtypeStruct((NC, 2, NP, DD), F32)
    stripe = NP // NS
    scratch = [pltpu.VMEM((K, BLK), jnp.int32), pltpu.VMEM((K, BLK), jnp.int32),
               pltpu.VMEM((BLK, DD), F32),
               pltpu.VMEM_SHARED((NP, DD), F32),
               pltpu.VMEM_SHARED((NP, DD), F32)]

    def body(src_h, dst_h, ones_h, zero_h, out_h, sidx, didx, ones_v, acc_s, acc_d):
        cid = lax.axis_index("c")
        sid = lax.axis_index("s")
        wid = cid * NS + sid
        sl = pl.ds(sid * stripe, stripe)
        pltpu.sync_copy(zero_h.at[sl], acc_s.at[sl])
        pltpu.sync_copy(zero_h.at[sl], acc_d.at[sl])
        pltpu.sync_copy(ones_h, ones_v)
        pltpu.sync_copy(src_h.at[wid], sidx)
        pltpu.sync_copy(dst_h.at[wid], didx)
        plsc.subcore_barrier()

        def step(j, _):
            pltpu.sync_copy(ones_v, acc_s.at[sidx.at[j]], add=True)
            pltpu.sync_copy(ones_v, acc_d.at[didx.at[j]], add=True)
            return 0

        lax.fori_loop(0, K, step, 0)
        plsc.subcore_barrier()
        pltpu.sync_copy(acc_s.at[sl], out_h.at[cid, 0, sl])
        pltpu.sync_copy(acc_d.at[sl], out_h.at[cid, 1, sl])

    return pl.kernel(body, out_type=out_type, mesh=_MESH, scratch_types=scratch)


def _leaky(x):
    return jnp.where(x >= 0, x, 0.2 * x)


def _tc1a(x, W1, AC):
    def body(x_r, w_r, ac_r, h_r, c_r):
        h = jnp.dot(x_r[...], w_r[...], preferred_element_type=F32)
        h_r[...] = h
        c_r[...] = jnp.dot(h, ac_r[...], preferred_element_type=F32)

    return pl.pallas_call(
        body,
        out_shape=[jax.ShapeDtypeStruct((N_S, 256), F32),
                   jax.ShapeDtypeStruct((N_S, 16), F32)],
    )(x, W1, AC)


def _tc1b(x, Wg1, degp):
    def body(x_r, w_r, d_r, h_r, din_r, dsrc_r):
        d = d_r[...]
        dout = jnp.maximum(d[:, 0:1] + d[:, 1:2], 1.0)
        din = jnp.maximum(d[:, 2:3] + d[:, 3:4], 1.0)
        dsrc = lax.rsqrt(dout)
        dinv = lax.rsqrt(din)
        h = jnp.dot(x_r[...], w_r[...], preferred_element_type=F32)
        h_r[...] = h * dsrc
        din_r[...] = jnp.broadcast_to(dinv, (N_L, 64))
        dsrc_r[...] = jnp.broadcast_to(dsrc, (N_L, 64))

    return pl.pallas_call(
        body,
        out_shape=[jax.ShapeDtypeStruct((N_L, 64), F32),
                   jax.ShapeDtypeStruct((N_L, 64), F32),
                   jax.ShapeDtypeStruct((N_L, 64), F32)],
    )(x, Wg1, degp)


def _tc2(s1a, s1b, dinB, dsrcB, bg1):
    def body(a_r, b_r, di_r, ds_r, bias_r, o_r):
        g = jax.nn.relu((a_r[...] + b_r[...]) * di_r[...] + bias_r[...])
        o_r[...] = g * ds_r[...]

    return pl.pallas_call(
        body, out_shape=jax.ShapeDtypeStruct((N_L, 64), F32),
    )(s1a, s1b, dinB, dsrcB, bg1)


def _tc3(s2a, s2b, dinB, bg2):
    def body(a_r, b_r, di_r, bias_r, o_r):
        o_r[...] = (a_r[...] + b_r[...]) * di_r[...] + bias_r[...]

    return pl.pallas_call(
        body, out_shape=jax.ShapeDtypeStruct((N_L, 64), F32),
    )(s2a, s2b, dinB, bg2)


def _tc_edge(a_src, a_dst, h_rows, rep, p16, heads):
    Ep, Dh = h_rows.shape
    Do = Dh + 16
    EB = 4096
    grid = (Ep // EB,)

    def body(as_r, ad_r, h_r, rep_r, p_r, o_r):
        w = jnp.exp(_leaky(as_r[:, 0:heads] + ad_r[:, 4:4 + heads]))
        wb = jnp.dot(w, rep_r[...], preferred_element_type=F32)
        wp = jnp.dot(w, p_r[...], preferred_element_type=F32)
        o_r[...] = jnp.concatenate([h_r[...] * wb, wp], axis=1)

    return pl.pallas_call(
        body,
        grid=grid,
        in_specs=[pl.BlockSpec((EB, 16), lambda i: (i, 0)),
                  pl.BlockSpec((EB, 16), lambda i: (i, 0)),
                  pl.BlockSpec((EB, Dh), lambda i: (i, 0)),
                  pl.BlockSpec((heads, Dh), lambda i: (0, 0)),
                  pl.BlockSpec((heads, 16), lambda i: (0, 0))],
        out_specs=pl.BlockSpec((EB, Do), lambda i: (i, 0)),
        out_shape=jax.ShapeDtypeStruct((Ep, Do), F32),
    )(a_src, a_dst, h_rows, rep, p16)


def _tc5(sa, sb, rep1, W2, AC2):
    def body(a_r, b_r, rep_r, w_r, ac_r, h2_r, c2_r):
        s = a_r[...] + b_r[...]
        den = jnp.dot(s[:, 256:260], rep_r[...], preferred_element_type=F32)
        gat1 = jax.nn.relu(s[:, 0:256] / (den + 1e-9))
        h2 = jnp.dot(gat1, w_r[...], preferred_element_type=F32)
        h2_r[...] = h2
        c2_r[...] = jnp.dot(h2, ac_r[...], preferred_element_type=F32)

    return pl.pallas_call(
        body,
        out_shape=[jax.ShapeDtypeStruct((N_S, 64), F32),
                   jax.ShapeDtypeStruct((N_S, 16), F32)],
    )(sa, sb, rep1, W2, AC2)


def _tc7(sa, sb, tl, Wc, bc):
    def body(a_r, b_r, t_r, w_r, bias_r, o_r):
        s = a_r[...] + b_r[...]
        gat2 = jax.nn.relu(s[:, 0:64] / (s[:, 64:65] + 1e-9))
        embs = jnp.concatenate([gat2, t_r[...]], axis=1)
        o_r[...] = jnp.dot(embs, w_r[...], preferred_element_type=F32) + bias_r[...]

    return pl.pallas_call(
        body, out_shape=jax.ShapeDtypeStruct((N_S, 32), F32),
    )(sa, sb, tl, Wc, bc)


def kernel(small_batch_embs, small_edge_index, token_idx_batch, large_embs,
           large_edge_index, W_gat1, al1, ar1, W_gat2, al2, ar2, Wg1, bg1,
           Wg2, bg2, Wc, bc):
    src_s, dst_s = small_edge_index[0], small_edge_index[1]
    src_l, dst_l = large_edge_index[0], large_edge_index[1]

    src_s_g = _pad_idx(src_s, 0, KS, BLK)
    dst_s_g = _pad_idx(dst_s, 0, KS, BLK)
    dst_s_s = _pad_idx(dst_s, N_S, KS, BLK)
    src_l_g = _pad_idx(src_l, 0, KL, BLK)
    src_l_s = _pad_idx(src_l, N_L, KL, BLK)
    dst_l_s = _pad_idx(dst_l, N_L, KL, BLK)
    tok = _pad_idx(token_idx_batch, 0, 1, 64)

    zeros_dd = jnp.zeros((NP_L, DD), F32)
    ones_dd = jnp.ones((BLK, DD), F32)
    zeros_l64 = jnp.zeros((NP_L, 64), F32)
    zeros_s272 = jnp.zeros((NP_S, 272), F32)
    zeros_s80 = jnp.zeros((NP_S, 80), F32)

    degp = _sc_degree(NP_L, KL)(src_l_s, dst_l_s, ones_dd, zeros_dd)
    degp4 = jnp.stack([degp[0, 0, :N_L, 0], degp[1, 0, :N_L, 0],
                       degp[0, 1, :N_L, 0], degp[1, 1, :N_L, 0]], axis=1)

    hL1s, dinB, dsrcB = _tc1b(large_embs, Wg1, degp4)
    S1 = _sc_segsum(64, NP_L, KL)(hL1s, src_l_g, dst_l_s, zeros_l64)
    g1s = _tc2(S1[0, :N_L], S1[1, :N_L], dinB, dsrcB, bg1.reshape(1, 64))
    S2 = _sc_segsum(64, NP_L, KL)(g1s, src_l_g, dst_l_s, zeros_l64)
    g2 = _tc3(S2[0, :N_L], S2[1, :N_L], dinB, bg2.reshape(1, 64))

    eye4 = jnp.eye(4, dtype=F32)
    AL1 = (eye4[:, None, :] * al1[:, :, None]).reshape(256, 4)
    AR1 = (eye4[:, None, :] * ar1[:, :, None]).reshape(256, 4)
    AC1 = jnp.concatenate([AL1, AR1, jnp.zeros((256, 8), F32)], axis=1)
    h1, C1 = _tc1a(small_batch_embs, W_gat1, AC1)

    g_spec1 = ((N_S, 16, KS, BLK), (N_S, 16, KS, BLK), (N_S, 256, KS, BLK))
    A1s, A1d, H1 = _sc_gather_multi(g_spec1)(C1, src_s_g, C1, dst_s_g, h1, src_s_g)
    REP1 = jnp.repeat(eye4, 64, axis=1)
    P16_1 = jnp.concatenate([eye4, jnp.zeros((4, 12), F32)], axis=1)
    M1 = _tc_edge(A1s, A1d, H1, REP1, P16_1, heads=4)
    Sm1 = _sc_scatter_add(272, NP_S, KS)(M1, dst_s_s, zeros_s272)

    AC2 = jnp.concatenate([al2.reshape(64, 1), jnp.zeros((64, 3), F32),
                           ar2.reshape(64, 1), jnp.zeros((64, 11), F32)], axis=1)
    h2, C2 = _tc5(Sm1[0, :N_S], Sm1[1, :N_S], REP1, W_gat2, AC2)

    g_spec2 = ((N_S, 16, KS, BLK), (N_S, 16, KS, BLK), (N_S, 64, KS, BLK),
               (N_L, 64, 1, 64))
    A2s, A2d, H2, tl = _sc_gather_multi(g_spec2)(
        C2, src_s_g, C2, dst_s_g, h2, src_s_g, g2, tok)
    REP2 = jnp.ones((1, 64), F32)
    P16_2 = jnp.concatenate([jnp.ones((1, 1), F32), jnp.zeros((1, 15), F32)], axis=1)
    M2 = _tc_edge(A2s, A2d, H2, REP2, P16_2, heads=1)
    Sm2 = _sc_scatter_add(80, NP_S, KS)(M2, dst_s_s, zeros_s80)

    return _tc7(Sm2[0, :N_S], Sm2[1, :N_S], tl[:N_S], Wc, bc.reshape(1, 32))


# speedup vs baseline: 10.7983x; 10.7983x over previous
import functools

import jax
import jax.numpy as jnp
from jax import lax
from jax.experimental import pallas as pl
from jax.experimental.pallas import tpu as pltpu
from jax.experimental.pallas import tpu_sc as plsc

F32 = jnp.float32
NC, NS = 2, 16
NW = NC * NS
BLK = 128
DD = 16

N_S, E_S = 2000, 32000
N_L, E_L = 10000, 320000
NP_S, NP_L = 2048, 10112
KS = 8
KL = 79

@functools.lru_cache(maxsize=None)
def _mesh():
    return plsc.VectorSubcoreMesh(core_axis_name="c", subcore_axis_name="s")


def _pad_idx(idx, fill, K, B):
    n = NW * K * B
    idx = idx.astype(jnp.int32)
    pad = jnp.full((n - idx.shape[0],), fill, jnp.int32)
    return jnp.concatenate([idx, pad]).reshape(NW, K, B)


@functools.lru_cache(maxsize=None)
def _sc_gather_multi(specs):
    n_ops = len(specs)
    out_type = [jax.ShapeDtypeStruct((NW * K * B, D), F32) for (_, D, K, B) in specs]
    scratch = []
    for (_, D, K, B) in specs:
        scratch += [pltpu.VMEM((K, B), jnp.int32), pltpu.VMEM((B, D), F32)]
    scratch.append(pltpu.SemaphoreType.DMA)

    def body(*refs):
        ins = refs[:2 * n_ops]
        outs = refs[2 * n_ops:3 * n_ops]
        scr = refs[3 * n_ops:]
        sem = scr[-1]
        cid = lax.axis_index("c")
        sid = lax.axis_index("s")
        wid = cid * NS + sid
        for i, (_, D, K, B) in enumerate(specs):
            table_h, idx_h = ins[2 * i], ins[2 * i + 1]
            out_h = outs[i]
            idx_v, rows_v = scr[2 * i], scr[2 * i + 1]
            pltpu.sync_copy(idx_h.at[wid], idx_v)

            def step(j, _, table_h=table_h, out_h=out_h, idx_v=idx_v,
                     rows_v=rows_v, K=K, B=B):
                pltpu.async_copy(table_h.at[idx_v.at[j]], rows_v, sem).wait()
                pltpu.sync_copy(rows_v, out_h.at[pl.ds((wid * K + j) * B, B)])
                return 0

            lax.fori_loop(0, K, step, 0)

    return pl.kernel(body, out_type=out_type, mesh=_mesh(), scratch_types=scratch,
                     compiler_params=pltpu.CompilerParams(use_tc_tiling_on_sc=False))


@functools.lru_cache(maxsize=None)
def _sc_segsum(D, NP, K):
    out_type = jax.ShapeDtypeStruct((NC, NP, D), F32)
    stripe = NP // NS
    scratch = [pltpu.VMEM((K, BLK), jnp.int32), pltpu.VMEM((K, BLK), jnp.int32),
               pltpu.VMEM((BLK, D), F32),
               pltpu.VMEM_SHARED((NP, D), F32),
               pltpu.SemaphoreType.DMA]

    def body(table_h, src_h, dst_h, zero_h, out_h, sidx, didx, rows, acc, sem):
        cid = lax.axis_index("c")
        sid = lax.axis_index("s")
        wid = cid * NS + sid
        sl = pl.ds(sid * stripe, stripe)
        pltpu.sync_copy(zero_h.at[sl], acc.at[sl])
        pltpu.sync_copy(src_h.at[wid], sidx)
        pltpu.sync_copy(dst_h.at[wid], didx)
        plsc.subcore_barrier()

        def step(j, _):
            pltpu.async_copy(table_h.at[sidx.at[j]], rows, sem).wait()
            pltpu.sync_copy(rows, acc.at[didx.at[j]], add=True)
            return 0

        lax.fori_loop(0, K, step, 0)
        plsc.subcore_barrier()
        pltpu.sync_copy(acc.at[sl], out_h.at[cid, sl])

    return pl.kernel(body, out_type=out_type, mesh=_mesh(), scratch_types=scratch,
                     compiler_params=pltpu.CompilerParams(use_tc_tiling_on_sc=False))


@functools.lru_cache(maxsize=None)
def _sc_scatter_add(D, NP, K):
    out_type = jax.ShapeDtypeStruct((NC, NP, D), F32)
    stripe = NP // NS
    scratch = [pltpu.VMEM((K, BLK), jnp.int32),
               pltpu.VMEM((BLK, D), F32),
               pltpu.VMEM_SHARED((NP, D), F32)]

    def body(vals_h, dst_h, zero_h, out_h, didx, rows, acc):
        cid = lax.axis_index("c")
        sid = lax.axis_index("s")
        wid = cid * NS + sid
        sl = pl.ds(sid * stripe, stripe)
        pltpu.sync_copy(zero_h.at[sl], acc.at[sl])
        pltpu.sync_copy(dst_h.at[wid], didx)
        plsc.subcore_barrier()

        def step(j, _):
            pltpu.sync_copy(vals_h.at[pl.ds((wid * K + j) * BLK, BLK)], rows)
            pltpu.sync_copy(rows, acc.at[didx.at[j]], add=True)
            return 0

        lax.fori_loop(0, K, step, 0)
        plsc.subcore_barrier()
        pltpu.sync_copy(acc.at[sl], out_h.at[cid, sl])

    return pl.kernel(body, out_type=out_type, mesh=_mesh(), scratch_types=scratch,
                     compiler_params=pltpu.CompilerParams(use_tc_tiling_on_sc=False))


@functools.lru_cache(maxsize=None)
def _sc_degree(NP, K):
    out_type = jax.ShapeDtypeStruct((NC, 2, NP, DD), F32)
    stripe = NP // NS
    scratch = [pltpu.VMEM((K, BLK), jnp.int32), pltpu.VMEM((K, BLK), jnp.int32),
               pltpu.VMEM((BLK, DD), F32),
               pltpu.VMEM_SHARED((NP, DD), F32),
               pltpu.VMEM_SHARED((NP, DD), F32)]

    def body(src_h, dst_h, ones_h, zero_h, out_h, sidx, didx, ones_v, acc_s, acc_d):
        cid = lax.axis_index("c")
        sid = lax.axis_index("s")
        wid = cid * NS + sid
        sl = pl.ds(sid * stripe, stripe)
        pltpu.sync_copy(zero_h.at[sl], acc_s.at[sl])
        pltpu.sync_copy(zero_h.at[sl], acc_d.at[sl])
        pltpu.sync_copy(ones_h, ones_v)
        pltpu.sync_copy(src_h.at[wid], sidx)
        pltpu.sync_copy(dst_h.at[wid], didx)
        plsc.subcore_barrier()

        def step(j, _):
            pltpu.sync_copy(ones_v, acc_s.at[sidx.at[j]], add=True)
            pltpu.sync_copy(ones_v, acc_d.at[didx.at[j]], add=True)
            return 0

        lax.fori_loop(0, K, step, 0)
        plsc.subcore_barrier()
        pltpu.sync_copy(acc_s.at[sl], out_h.at[cid, 0, sl])
        pltpu.sync_copy(acc_d.at[sl], out_h.at[cid, 1, sl])

    return pl.kernel(body, out_type=out_type, mesh=_mesh(), scratch_types=scratch,
                     compiler_params=pltpu.CompilerParams(use_tc_tiling_on_sc=False))


def _leaky(x):
    return jnp.where(x >= 0, x, 0.2 * x)


def _tc1a(x, W1, AC):
    def body(x_r, w_r, ac_r, h_r, c_r):
        h = jnp.dot(x_r[...], w_r[...], preferred_element_type=F32)
        h_r[...] = h
        c_r[...] = jnp.dot(h, ac_r[...], preferred_element_type=F32)

    return pl.pallas_call(
        body,
        out_shape=[jax.ShapeDtypeStruct((N_S, 256), F32),
                   jax.ShapeDtypeStruct((N_S, 16), F32)],
    )(x, W1, AC)


def _tc1b(x, Wg1, degp):
    def body(x_r, w_r, d_r, h_r, din_r, dsrc_r):
        d = d_r[...]
        dout = jnp.maximum(d[:, 0:1] + d[:, 1:2], 1.0)
        din = jnp.maximum(d[:, 2:3] + d[:, 3:4], 1.0)
        dsrc = lax.rsqrt(dout)
        dinv = lax.rsqrt(din)
        h = jnp.dot(x_r[...], w_r[...], preferred_element_type=F32)
        h_r[...] = h * dsrc
        din_r[...] = jnp.broadcast_to(dinv, (N_L, 64))
        dsrc_r[...] = jnp.broadcast_to(dsrc, (N_L, 64))

    return pl.pallas_call(
        body,
        out_shape=[jax.ShapeDtypeStruct((N_L, 64), F32),
                   jax.ShapeDtypeStruct((N_L, 64), F32),
                   jax.ShapeDtypeStruct((N_L, 64), F32)],
    )(x, Wg1, degp)


def _tc2(s1a, s1b, dinB, dsrcB, bg1, Wg2):
    def body(a_r, b_r, di_r, ds_r, bias_r, w_r, o_r):
        g = jax.nn.relu((a_r[...] + b_r[...]) * di_r[...] + bias_r[...])
        o_r[...] = jnp.dot(g, w_r[...], preferred_element_type=F32) * ds_r[...]

    return pl.pallas_call(
        body, out_shape=jax.ShapeDtypeStruct((N_L, 64), F32),
    )(s1a, s1b, dinB, dsrcB, bg1, Wg2)


def _tc3(s2a, s2b, dinB, bg2):
    def body(a_r, b_r, di_r, bias_r, o_r):
        o_r[...] = (a_r[...] + b_r[...]) * di_r[...] + bias_r[...]

    return pl.pallas_call(
        body, out_shape=jax.ShapeDtypeStruct((N_L, 64), F32),
    )(s2a, s2b, dinB, bg2)


def _tc_edge(a_src, a_dst, h_rows, rep, p16, heads):
    Ep, Dh = h_rows.shape
    Do = Dh + 16
    EB = 4096
    grid = (Ep // EB,)

    def body(as_r, ad_r, h_r, rep_r, p_r, o_r):
        w = jnp.exp(_leaky(as_r[:, 0:heads] + ad_r[:, 4:4 + heads]))
        wb = jnp.dot(w, rep_r[...], preferred_element_type=F32)
        wp = jnp.dot(w, p_r[...], preferred_element_type=F32)
        o_r[...] = jnp.concatenate([h_r[...] * wb, wp], axis=1)

    return pl.pallas_call(
        body,
        grid=grid,
        in_specs=[pl.BlockSpec((EB, 16), lambda i: (i, 0)),
                  pl.BlockSpec((EB, 16), lambda i: (i, 0)),
                  pl.BlockSpec((EB, Dh), lambda i: (i, 0)),
                  pl.BlockSpec((heads, Dh), lambda i: (0, 0)),
                  pl.BlockSpec((heads, 16), lambda i: (0, 0))],
        out_specs=pl.BlockSpec((EB, Do), lambda i: (i, 0)),
        out_shape=jax.ShapeDtypeStruct((Ep, Do), F32),
    )(a_src, a_dst, h_rows, rep, p16)


def _tc5(sa, sb, rep1, W2, AC2):
    def body(a_r, b_r, rep_r, w_r, ac_r, h2_r, c2_r):
        s = a_r[...] + b_r[...]
        den = jnp.dot(s[:, 256:260], rep_r[...], preferred_element_type=F32)
        gat1 = jax.nn.relu(s[:, 0:256] / (den + 1e-9))
        h2 = jnp.dot(gat1, w_r[...], preferred_element_type=F32)
        h2_r[...] = h2
        c2_r[...] = jnp.dot(h2, ac_r[...], preferred_element_type=F32)

    return pl.pallas_call(
        body,
        out_shape=[jax.ShapeDtypeStruct((N_S, 64), F32),
                   jax.ShapeDtypeStruct((N_S, 16), F32)],
    )(sa, sb, rep1, W2, AC2)


def _tc7(sa, sb, tl, Wc, bc):
    def body(a_r, b_r, t_r, w_r, bias_r, o_r):
        s = a_r[...] + b_r[...]
        gat2 = jax.nn.relu(s[:, 0:64] / (s[:, 64:65] + 1e-9))
        embs = jnp.concatenate([gat2, t_r[...]], axis=1)
        o_r[...] = jnp.dot(embs, w_r[...], preferred_element_type=F32) + bias_r[...]

    return pl.pallas_call(
        body, out_shape=jax.ShapeDtypeStruct((N_S, 32), F32),
    )(sa, sb, tl, Wc, bc)


def kernel(small_batch_embs, small_edge_index, token_idx_batch, large_embs,
           large_edge_index, W_gat1, al1, ar1, W_gat2, al2, ar2, Wg1, bg1,
           Wg2, bg2, Wc, bc):
    src_s, dst_s = small_edge_index[0], small_edge_index[1]
    src_l, dst_l = large_edge_index[0], large_edge_index[1]

    src_s_g = _pad_idx(src_s, 0, KS, BLK)
    dst_s_g = _pad_idx(dst_s, 0, KS, BLK)
    dst_s_s = _pad_idx(dst_s, N_S, KS, BLK)
    src_l_g = _pad_idx(src_l, 0, KL, BLK)
    src_l_s = _pad_idx(src_l, N_L, KL, BLK)
    dst_l_s = _pad_idx(dst_l, N_L, KL, BLK)
    tok = _pad_idx(token_idx_batch, 0, 1, 64)

    zeros_dd = jnp.zeros((NP_L, DD), F32)
    ones_dd = jnp.ones((BLK, DD), F32)
    zeros_l64 = jnp.zeros((NP_L, 64), F32)
    zeros_s272 = jnp.zeros((NP_S, 272), F32)
    zeros_s80 = jnp.zeros((NP_S, 80), F32)

    degp = _sc_degree(NP_L, KL)(src_l_s, dst_l_s, ones_dd, zeros_dd)
    degp4 = jnp.stack([degp[0, 0, :N_L, 0], degp[1, 0, :N_L, 0],
                       degp[0, 1, :N_L, 0], degp[1, 1, :N_L, 0]], axis=1)

    hL1s, dinB, dsrcB = _tc1b(large_embs, Wg1, degp4)
    S1 = _sc_segsum(64, NP_L, KL)(hL1s, src_l_g, dst_l_s, zeros_l64)
    g1s = _tc2(S1[0, :N_L], S1[1, :N_L], dinB, dsrcB, bg1.reshape(1, 64), Wg2)
    S2 = _sc_segsum(64, NP_L, KL)(g1s, src_l_g, dst_l_s, zeros_l64)
    g2 = _tc3(S2[0, :N_L], S2[1, :N_L], dinB, bg2.reshape(1, 64))

    eye4 = jnp.eye(4, dtype=F32)
    AL1 = (eye4[:, None, :] * al1[:, :, None]).reshape(256, 4)
    AR1 = (eye4[:, None, :] * ar1[:, :, None]).reshape(256, 4)
    AC1 = jnp.concatenate([AL1, AR1, jnp.zeros((256, 8), F32)], axis=1)
    h1, C1 = _tc1a(small_batch_embs, W_gat1, AC1)

    g_spec1 = ((N_S, 16, KS, BLK), (N_S, 16, KS, BLK), (N_S, 256, KS, BLK))
    A1s, A1d, H1 = _sc_gather_multi(g_spec1)(C1, src_s_g, C1, dst_s_g, h1, src_s_g)
    REP1 = jnp.repeat(eye4, 64, axis=1)
    P16_1 = jnp.concatenate([eye4, jnp.zeros((4, 12), F32)], axis=1)
    M1 = _tc_edge(A1s, A1d, H1, REP1, P16_1, heads=4)
    Sm1 = _sc_scatter_add(272, NP_S, KS)(M1, dst_s_s, zeros_s272)

    AC2 = jnp.concatenate([al2.reshape(64, 1), jnp.zeros((64, 3), F32),
                           ar2.reshape(64, 1), jnp.zeros((64, 11), F32)], axis=1)
    h2, C2 = _tc5(Sm1[0, :N_S], Sm1[1, :N_S], REP1, W_gat2, AC2)

    g_spec2 = ((N_S, 16, KS, BLK), (N_S, 16, KS, BLK), (N_S, 64, KS, BLK),
               (N_L, 64, 1, 64))
    A2s, A2d, H2, tl = _sc_gather_multi(g_spec2)(
        C2, src_s_g, C2, dst_s_g, h2, src_s_g, g2, tok)
    REP2 = jnp.ones((1, 64), F32)
    P16_2 = jnp.concatenate([jnp.ones((1, 1), F32), jnp.zeros((1, 15), F32)], axis=1)
    M2 = _tc_edge(A2s, A2d, H2, REP2, P16_2, heads=1)
    Sm2 = _sc_scatter_add(80, NP_S, KS)(M2, dst_s_s, zeros_s80)

    return _tc7(Sm2[0, :N_S], Sm2[1, :N_S], tl[:N_S], Wc, bc.reshape(1, 32))
